# Initial kernel scaffold; baseline (speedup 1.0000x reference)
import jax
import jax.numpy as jnp
from jax.experimental import pallas as pl


def kernel(output, mask, ind, target):
    raise NotImplementedError("write your pallas kernel here")


# speedup vs baseline: 1.3743x; 1.3743x over previous
import jax
import jax.numpy as jnp
from jax import lax
from jax.experimental import pallas as pl
from jax.experimental.pallas import tpu as pltpu


def _loss_kernel(ind_ref, out_ref, mask_ref, tgt_ref, loss_ref, acc_ref):
    b = pl.program_id(0)
    B = pl.num_programs(0)

    @pl.when(b == 0)
    def _():
        acc_ref[0] = 0.0
        acc_ref[1] = 0.0

    feat = out_ref[0].reshape(64, 128 * 128)
    indv = ind_ref[0]

    CH = 2048
    acc = jnp.zeros((512, 64), dtype=jnp.float32)
    for j in range(128 * 128 // CH):
        iota = lax.broadcasted_iota(jnp.int32, (512, CH), 1) + j * CH
        oh = (indv == iota).astype(jnp.float32)
        fc = feat[:, j * CH:(j + 1) * CH]
        acc = acc + lax.dot_general(
            oh, fc, (((1,), (1,)), ((), ())),
            preferred_element_type=jnp.float32)

    m = mask_ref[0]
    t = tgt_ref[0]
    num_b = jnp.sum(jnp.abs(acc * m - t * m))
    den_b = jnp.sum(m)
    acc_ref[0] = acc_ref[0] + num_b
    acc_ref[1] = acc_ref[1] + den_b

    @pl.when(b == B - 1)
    def _():
        loss_ref[...] = (acc_ref[0] / (acc_ref[1] + 0.0001))[None, None]


def kernel(output, mask, ind, target):
    B, C, H, W = output.shape
    K = ind.shape[1]
    KP = 512
    ind_p = jnp.pad(ind, ((0, 0), (0, KP - K))).reshape(B, KP, 1)
    mask_p = jnp.pad(mask, ((0, 0), (0, KP - K), (0, 0)))
    tgt_p = jnp.pad(target, ((0, 0), (0, KP - K), (0, 0)))

    loss = pl.pallas_call(
        _loss_kernel,
        grid=(B,),
        in_specs=[
            pl.BlockSpec((1, KP, 1), lambda b: (b, 0, 0)),
            pl.BlockSpec((1, C, H, W), lambda b: (b, 0, 0, 0)),
            pl.BlockSpec((1, KP, C), lambda b: (b, 0, 0)),
            pl.BlockSpec((1, KP, C), lambda b: (b, 0, 0)),
        ],
        out_specs=pl.BlockSpec((1, 1), lambda b: (0, 0)),
        out_shape=jax.ShapeDtypeStruct((1, 1), jnp.float32),
        scratch_shapes=[pltpu.SMEM((2,), jnp.float32)],
    )(ind_p, output, mask_p, tgt_p)
    return loss[0, 0]


# speedup vs baseline: 2.5398x; 1.8480x over previous
import jax
import jax.numpy as jnp
from jax import lax
from jax.experimental import pallas as pl
from jax.experimental.pallas import tpu as pltpu
from jax.experimental.pallas import tpu_sc as plsc

B, C, HW = 32, 64, 128 * 128
K, KP = 500, 512
P = 128
NCHUNK = KP // P


def _sc_body(out_hbm, ind_hbm, mask_hbm, tgt_hbm, part_hbm,
             ind_v, idx_v, pred_v, mask_v, tgt_v, part_v, gsem, lsem):
    nc = 2
    wid = lax.axis_index("s") * nc + lax.axis_index("c")
    b = wid

    lane = lax.broadcasted_iota(jnp.int32, (16,), 0)
    civ = [(lane + j * 16) * HW for j in range(C // 16)]

    pltpu.sync_copy(ind_hbm.at[b], ind_v.at[pl.ds(0, KP)])

    acc = jnp.zeros((16,), jnp.float32)
    msum = jnp.zeros((16,), jnp.float32)

    for chunk in range(NCHUNK):
        k0 = chunk * P
        n = min(K - k0, P)

        mcopy = pltpu.async_copy(
            mask_hbm.at[b, pl.ds(k0 * C, n * C)], mask_v.at[pl.ds(0, n * C)],
            lsem)
        tcopy = pltpu.async_copy(
            tgt_hbm.at[b, pl.ds(k0 * C, n * C)], tgt_v.at[pl.ds(0, n * C)],
            lsem)

        def build(p, _):
            base = b * (C * HW) + ind_v[pl.ds(k0 + p, 16)][0]
            for j in range(C // 16):
                idx_v[p, pl.ds(j * 16, 16)] = civ[j] + base
            return 0
        lax.fori_loop(0, P, build, 0, unroll=4)

        def fire(g, _):
            for j in range(8):
                row = g * 8 + j
                pltpu.async_copy(out_hbm.at[idx_v.at[row]],
                                 pred_v.at[pl.ds(row * C, C)], gsem)
            return 0
        lax.fori_loop(0, P // 8, fire, 0)

        mcopy.wait()
        tcopy.wait()

        def drain(g, _):
            for j in range(8):
                row = g * 8 + j
                pltpu.make_async_copy(out_hbm.at[idx_v.at[row]],
                                      pred_v.at[pl.ds(row * C, C)],
                                      gsem).wait()
            return 0
        lax.fori_loop(0, P // 8, drain, 0)

        def red(v, carry):
            a, m = carry
            for j in range(4):
                o = (v * 4 + j) * 16
                mv = mask_v[pl.ds(o, 16)]
                tv = tgt_v[pl.ds(o, 16)]
                pv = pred_v[pl.ds(o, 16)]
                a = a + jnp.abs(pv * mv - tv * mv)
                m = m + mv
            return a, m
        acc, msum = lax.fori_loop(0, n * C // 64, red, (acc, msum), unroll=4)

    part_v[0, :] = acc
    part_v[1, :] = msum
    pltpu.sync_copy(part_v, part_hbm.at[wid])


@jax.jit
def _sc_loss(output, mask, ind, target):
    out_flat = output.reshape(B * C * HW)
    ind_p = jnp.pad(ind, ((0, 0), (0, KP - K)))
    mask2 = mask.reshape(B, K * C)
    target2 = target.reshape(B, K * C)
    mesh = plsc.VectorSubcoreMesh(core_axis_name="c", subcore_axis_name="s")
    parts = pl.kernel(
        _sc_body,
        out_type=jax.ShapeDtypeStruct((B, 2, 16), jnp.float32),
        mesh=mesh,
        scratch_types=[
            pltpu.VMEM((KP + 16,), jnp.int32),
            pltpu.VMEM((P, C), jnp.int32),
            pltpu.VMEM((P * C,), jnp.float32),
            pltpu.VMEM((P * C,), jnp.float32),
            pltpu.VMEM((P * C,), jnp.float32),
            pltpu.VMEM((2, 16), jnp.float32),
            pltpu.SemaphoreType.DMA,
            pltpu.SemaphoreType.DMA,
        ],
    )(out_flat, ind_p, mask2, target2)
    num = jnp.sum(parts[:, 0, :])
    den = jnp.sum(parts[:, 1, :])
    return num / (den + 0.0001)


def kernel(output, mask, ind, target):
    return _sc_loss(output, mask, ind, target)


# speedup vs baseline: 3.2214x; 1.2684x over previous
import jax
import jax.numpy as jnp
from jax import lax
from jax.experimental import pallas as pl
from jax.experimental.pallas import tpu as pltpu
from jax.experimental.pallas import tpu_sc as plsc

B, C, HW = 32, 64, 128 * 128
K, KP = 500, 512
P = 128
NCHUNK = KP // P
NROW = P // 2


def _sc_body(out_hbm, ind_hbm, mask_hbm, tgt_hbm, part_hbm,
             ind_v, idx_v, pred_v, mask_v, tgt_v, part_v,
             gsem0, gsem1, lsem0, lsem1):
    gsems = (gsem0, gsem1)
    lsems = (lsem0, lsem1)
    nc = 2
    wid = lax.axis_index("s") * nc + lax.axis_index("c")
    b = wid

    lane = lax.broadcasted_iota(jnp.int32, (16,), 0)
    civ = [(lane + j * 16) * HW for j in range(C // 16)]

    pltpu.sync_copy(ind_hbm.at[b], ind_v.at[pl.ds(0, KP)])

    def valid(chunk):
        return min(K - chunk * P, P)

    def fire_linear(chunk):
        pb = chunk % 2
        n = valid(chunk)
        m = pltpu.async_copy(mask_hbm.at[b, pl.ds(chunk * P, n)],
                             mask_v.at[pb, pl.ds(0, n)], lsems[pb])
        t = pltpu.async_copy(tgt_hbm.at[b, pl.ds(chunk * P, n)],
                             tgt_v.at[pb, pl.ds(0, n)], lsems[pb])
        return m, t

    def build(chunk):
        pb = chunk % 2
        k0 = chunk * P

        def body(r, _):
            b0 = b * (C * HW) + ind_v[pl.ds(k0 + 2 * r, 16)][0]
            b1 = b * (C * HW) + ind_v[pl.ds(k0 + 2 * r + 1, 16)][0]
            for j in range(C // 16):
                idx_v[pb, r, pl.ds(j * 16, 16)] = civ[j] + b0
                idx_v[pb, r, pl.ds(C + j * 16, 16)] = civ[j] + b1
            return 0
        lax.fori_loop(0, NROW, body, 0, unroll=4)

    def fire_gathers(chunk):
        pb = chunk % 2

        def body(g, _):
            for j in range(8):
                row = g * 8 + j
                pltpu.async_copy(out_hbm.at[idx_v.at[pb, row]],
                                 pred_v.at[pb, pl.ds(row * 2 * C, 2 * C)],
                                 gsems[pb])
            return 0
        lax.fori_loop(0, NROW // 8, body, 0)

    def drain_gathers(chunk):
        pb = chunk % 2

        def body(g, _):
            for j in range(8):
                row = g * 8 + j
                pltpu.make_async_copy(out_hbm.at[idx_v.at[pb, row]],
                                      pred_v.at[pb, pl.ds(row * 2 * C, 2 * C)],
                                      gsems[pb]).wait()
            return 0
        lax.fori_loop(0, NROW // 8, body, 0)

    acc = jnp.zeros((16,), jnp.float32)
    msum = jnp.zeros((16,), jnp.float32)

    fire_linear(0)
    build(0)
    fire_gathers(0)
    for chunk in range(NCHUNK):
        if chunk + 1 < NCHUNK:
            fire_linear(chunk + 1)
            build(chunk + 1)
            fire_gathers(chunk + 1)
        pb = chunk % 2
        n = valid(chunk)
        m, t = (pltpu.make_async_copy(
                    mask_hbm.at[b, pl.ds(chunk * P, n)],
                    mask_v.at[pb, pl.ds(0, n)], lsems[pb]),
                pltpu.make_async_copy(
                    tgt_hbm.at[b, pl.ds(chunk * P, n)],
                    tgt_v.at[pb, pl.ds(0, n)], lsems[pb]))
        m.wait()
        t.wait()
        drain_gathers(chunk)

        def body(p, carry):
            a, ms = carry
            for j in range(C // 16):
                mv = mask_v[pb, p, pl.ds(j * 16, 16)]
                tv = tgt_v[pb, p, pl.ds(j * 16, 16)]
                pv = pred_v[pb, pl.ds(p * C + j * 16, 16)]
                a = a + jnp.abs(pv * mv - tv * mv)
                ms = ms + mv
            return a, ms
        acc, msum = lax.fori_loop(0, n, body, (acc, msum), unroll=4)

    part_v[0, :] = acc
    part_v[1, :] = msum
    pltpu.sync_copy(part_v, part_hbm.at[wid])


@jax.jit
def _sc_loss(output, mask, ind, target):
    out_flat = output.reshape(B * C * HW)
    ind_p = jnp.pad(ind, ((0, 0), (0, KP - K)))
    mesh = plsc.VectorSubcoreMesh(core_axis_name="c", subcore_axis_name="s")
    parts = pl.kernel(
        _sc_body,
        out_type=jax.ShapeDtypeStruct((B, 2, 16), jnp.float32),
        mesh=mesh,
        scratch_types=[
            pltpu.VMEM((KP + 16,), jnp.int32),
            pltpu.VMEM((2, NROW, 2 * C), jnp.int32),
            pltpu.VMEM((2, P * C), jnp.float32),
            pltpu.VMEM((2, P, C), jnp.float32),
            pltpu.VMEM((2, P, C), jnp.float32),
            pltpu.VMEM((2, 16), jnp.float32),
            pltpu.SemaphoreType.DMA,
            pltpu.SemaphoreType.DMA,
            pltpu.SemaphoreType.DMA,
            pltpu.SemaphoreType.DMA,
        ],
    )(out_flat, ind_p, mask, target)
    num = jnp.sum(parts[:, 0, :])
    den = jnp.sum(parts[:, 1, :])
    return num / (den + 0.0001)


def kernel(output, mask, ind, target):
    return _sc_loss(output, mask, ind, target)
